# Initial kernel scaffold; baseline (speedup 1.0000x reference)
import functools

import jax
import jax.numpy as jnp
from jax import lax
from jax.experimental import pallas as pl
from jax.experimental.pallas import tpu as pltpu
from jax.experimental.pallas import tpu_sc as plsc

B = 4096
N_HIST = 20
CAP_LEN = 50
TOK = N_HIST + CAP_LEN
ACT_VOCAB = 1000
CAP_VOCAB = 100000
DIM = 128
ROWS = B * TOK

_PROJ_BLK = 1000

_NC = 2
_NS = 16
_NW = _NC * _NS
_RPW = ROWS // _NW
_GRP = _RPW // 128


def _proj_body(a_ref, wa_ref, ba_ref, c_ref, wc_ref, bc_ref, o_ref):
    i = pl.program_id(0)

    @pl.when(i == 0)
    def _():
        o_ref[...] = (
            jnp.dot(a_ref[...], wa_ref[...], preferred_element_type=jnp.float32)
            + ba_ref[...]
        )

    @pl.when(i > 0)
    def _():
        o_ref[...] = (
            jnp.dot(c_ref[...], wc_ref[...], preferred_element_type=jnp.float32)
            + bc_ref[...]
        )


def _project_tables(action_emb, Wa, ba, caption_emb, Wc, bc):
    n_cap_blocks = CAP_VOCAB // _PROJ_BLK
    grid = (1 + n_cap_blocks,)
    return pl.pallas_call(
        _proj_body,
        grid=grid,
        in_specs=[
            pl.BlockSpec((ACT_VOCAB, DIM), lambda i: (0, 0)),
            pl.BlockSpec((DIM, DIM), lambda i: (0, 0)),
            pl.BlockSpec((1, DIM), lambda i: (0, 0)),
            pl.BlockSpec((_PROJ_BLK, DIM), lambda i: (jnp.maximum(i - 1, 0), 0)),
            pl.BlockSpec((DIM, DIM), lambda i: (0, 0)),
            pl.BlockSpec((1, DIM), lambda i: (0, 0)),
        ],
        out_specs=pl.BlockSpec((_PROJ_BLK, DIM), lambda i: (i, 0)),
        out_shape=jax.ShapeDtypeStruct((ACT_VOCAB + CAP_VOCAB, DIM), jnp.float32),
    )(action_emb, Wa, ba.reshape(1, DIM), caption_emb, Wc, bc.reshape(1, DIM))


def _gather_rows(ptab, idx2d):
    mesh = plsc.VectorSubcoreMesh(core_axis_name="c", subcore_axis_name="s")

    @functools.partial(
        pl.kernel,
        mesh=mesh,
        out_type=jax.ShapeDtypeStruct((ROWS, DIM), jnp.float32),
        scratch_types=[
            pltpu.VMEM((_GRP, 128), jnp.int32),
            pltpu.VMEM((128, DIM), jnp.float32),
            pltpu.SemaphoreType.DMA,
        ],
    )
    def k(p_hbm, idx_hbm, out_hbm, idx_v, buf, sem):
        wid = lax.axis_index("s") * _NC + lax.axis_index("c")
        pltpu.sync_copy(idx_hbm.at[pl.ds(wid * _GRP, _GRP)], idx_v)

        def body(g, carry):
            pltpu.async_copy(p_hbm.at[idx_v.at[g]], buf, sem).wait()
            pltpu.sync_copy(buf, out_hbm.at[pl.ds(wid * _RPW + g * 128, 128)])
            return carry

        lax.fori_loop(0, _GRP, body, 0)

    return k(ptab, idx2d)


def kernel(actions, captions, action_emb, Wa, ba, caption_emb, Wc, bc):
    ptab = _project_tables(action_emb, Wa, ba, caption_emb, Wc, bc)
    idx = jnp.concatenate([actions, captions + ACT_VOCAB], axis=1)
    idx2d = idx.reshape(ROWS // 128, 128)
    enc = _gather_rows(ptab, idx2d).reshape(B, TOK, DIM)
    obs_mask = jnp.concatenate(
        [jnp.zeros((B, N_HIST), dtype=bool), jnp.ones((B, CAP_LEN), dtype=bool)],
        axis=1,
    )
    return enc, obs_mask


# speedup vs baseline: 2.5472x; 2.5472x over previous
import functools

import jax
import jax.numpy as jnp
from jax import lax
from jax.experimental import pallas as pl
from jax.experimental.pallas import tpu as pltpu
from jax.experimental.pallas import tpu_sc as plsc

B = 4096
N_HIST = 20
CAP_LEN = 50
TOK = N_HIST + CAP_LEN
ACT_VOCAB = 1000
CAP_VOCAB = 100000
DIM = 128
ROWS = B * TOK

_PROJ_BLK = 1000

_NC = 2
_NS = 16
_NW = _NC * _NS
_RPW = ROWS // _NW
_GRP = _RPW // 128


def _proj_body(a_ref, wa_ref, ba_ref, c_ref, wc_ref, bc_ref, o_ref):
    i = pl.program_id(0)

    @pl.when(i == 0)
    def _():
        o_ref[...] = (
            jnp.dot(a_ref[...], wa_ref[...], preferred_element_type=jnp.float32)
            + ba_ref[...]
        )

    @pl.when(i > 0)
    def _():
        o_ref[...] = (
            jnp.dot(c_ref[...], wc_ref[...], preferred_element_type=jnp.float32)
            + bc_ref[...]
        )


def _project_tables(action_emb, Wa, ba, caption_emb, Wc, bc):
    n_cap_blocks = CAP_VOCAB // _PROJ_BLK
    grid = (1 + n_cap_blocks,)
    return pl.pallas_call(
        _proj_body,
        grid=grid,
        in_specs=[
            pl.BlockSpec((ACT_VOCAB, DIM), lambda i: (0, 0)),
            pl.BlockSpec((DIM, DIM), lambda i: (0, 0)),
            pl.BlockSpec((1, DIM), lambda i: (0, 0)),
            pl.BlockSpec((_PROJ_BLK, DIM), lambda i: (jnp.maximum(i - 1, 0), 0)),
            pl.BlockSpec((DIM, DIM), lambda i: (0, 0)),
            pl.BlockSpec((1, DIM), lambda i: (0, 0)),
        ],
        out_specs=pl.BlockSpec((_PROJ_BLK, DIM), lambda i: (i, 0)),
        out_shape=jax.ShapeDtypeStruct((ACT_VOCAB + CAP_VOCAB, DIM), jnp.float32),
    )(action_emb, Wa, ba.reshape(1, DIM), caption_emb, Wc, bc.reshape(1, DIM))


def _gather_rows(ptab, idx2d):
    mesh = plsc.VectorSubcoreMesh(core_axis_name="c", subcore_axis_name="s")

    @functools.partial(
        pl.kernel,
        mesh=mesh,
        out_type=jax.ShapeDtypeStruct((ROWS, DIM), jnp.float32),
        scratch_types=[
            pltpu.VMEM((_GRP, 128), jnp.int32),
            pltpu.VMEM((128, DIM), jnp.float32),
            pltpu.SemaphoreType.DMA,
        ],
    )
    def k(p_hbm, idx_hbm, out_hbm, idx_v, buf, sem):
        wid = lax.axis_index("s") * _NC + lax.axis_index("c")
        pltpu.sync_copy(idx_hbm.at[wid], idx_v)

        def body(g, carry):
            pltpu.async_copy(p_hbm.at[idx_v.at[g]], buf, sem).wait()
            pltpu.sync_copy(buf, out_hbm.at[pl.ds(wid * _RPW + g * 128, 128)])
            return carry

        lax.fori_loop(0, _GRP, body, 0)

    return k(ptab, idx2d)


def kernel(actions, captions, action_emb, Wa, ba, caption_emb, Wc, bc):
    ptab = _project_tables(action_emb, Wa, ba, caption_emb, Wc, bc)
    idx = jnp.concatenate([actions, captions + ACT_VOCAB], axis=1)
    idx3d = idx.reshape(_NW, _GRP, 128)
    enc = _gather_rows(ptab, idx3d).reshape(B, TOK, DIM)
    obs_mask = jnp.concatenate(
        [jnp.zeros((B, N_HIST), dtype=bool), jnp.ones((B, CAP_LEN), dtype=bool)],
        axis=1,
    )
    return enc, obs_mask


# speedup vs baseline: 4.4027x; 1.7285x over previous
import functools

import jax
import jax.numpy as jnp
from jax import lax
from jax.experimental import pallas as pl
from jax.experimental.pallas import tpu as pltpu
from jax.experimental.pallas import tpu_sc as plsc

B = 4096
N_HIST = 20
CAP_LEN = 50
TOK = N_HIST + CAP_LEN
ACT_VOCAB = 1000
CAP_VOCAB = 100000
DIM = 128
ROWS = B * TOK

_PROJ_BLK = 1000

_NC = 2
_NS = 16
_NW = _NC * _NS
_BPW = B // _NW
_NBUF = 8
_DIST = 4


def _proj_body(a_ref, wa_ref, ba_ref, c_ref, wc_ref, bc_ref, o_ref):
    i = pl.program_id(0)

    @pl.when(i == 0)
    def _():
        o_ref[...] = (
            jnp.dot(a_ref[...], wa_ref[...], preferred_element_type=jnp.float32)
            + ba_ref[...]
        )

    @pl.when(i > 0)
    def _():
        o_ref[...] = (
            jnp.dot(c_ref[...], wc_ref[...], preferred_element_type=jnp.float32)
            + bc_ref[...]
        )


def _project_tables(action_emb, Wa, ba, caption_emb, Wc, bc):
    n_cap_blocks = CAP_VOCAB // _PROJ_BLK
    grid = (1 + n_cap_blocks,)
    return pl.pallas_call(
        _proj_body,
        grid=grid,
        in_specs=[
            pl.BlockSpec((ACT_VOCAB, DIM), lambda i: (0, 0)),
            pl.BlockSpec((DIM, DIM), lambda i: (0, 0)),
            pl.BlockSpec((1, DIM), lambda i: (0, 0)),
            pl.BlockSpec((_PROJ_BLK, DIM), lambda i: (jnp.maximum(i - 1, 0), 0)),
            pl.BlockSpec((DIM, DIM), lambda i: (0, 0)),
            pl.BlockSpec((1, DIM), lambda i: (0, 0)),
        ],
        out_specs=pl.BlockSpec((_PROJ_BLK, DIM), lambda i: (i, 0)),
        out_shape=jax.ShapeDtypeStruct((ACT_VOCAB + CAP_VOCAB, DIM), jnp.float32),
    )(action_emb, Wa, ba.reshape(1, DIM), caption_emb, Wc, bc.reshape(1, DIM))


def _gather_rows(ptab, idx3d):
    mesh = plsc.VectorSubcoreMesh(core_axis_name="c", subcore_axis_name="s")

    @functools.partial(
        pl.kernel,
        mesh=mesh,
        out_type=jax.ShapeDtypeStruct((B, TOK, DIM), jnp.float32),
        scratch_types=[
            pltpu.VMEM((_BPW, TOK), jnp.int32),
            pltpu.VMEM((_NBUF, TOK, DIM), jnp.float32),
            pltpu.SemaphoreType.DMA,
            pltpu.SemaphoreType.DMA,
        ],
    )
    def k(p_hbm, idx_hbm, out_hbm, idx_v, bufs, gsem, osem):
        wid = lax.axis_index("s") * _NC + lax.axis_index("c")
        obase = wid * _BPW
        pltpu.sync_copy(idx_hbm.at[wid], idx_v)

        for t in range(_DIST):
            pltpu.make_async_copy(
                p_hbm.at[idx_v.at[t]], bufs.at[t], gsem
            ).start()

        def body(i, carry):
            @pl.when(i >= _DIST)
            def _():
                pltpu.make_async_copy(
                    bufs.at[(i - _DIST) % _NBUF],
                    out_hbm.at[obase + i - _DIST],
                    osem,
                ).wait()

            @pl.when(i < _BPW - _DIST)
            def _():
                pltpu.make_async_copy(
                    p_hbm.at[idx_v.at[i + _DIST]],
                    bufs.at[(i + _DIST) % _NBUF],
                    gsem,
                ).start()

            pltpu.make_async_copy(
                p_hbm.at[idx_v.at[i]], bufs.at[i % _NBUF], gsem
            ).wait()
            pltpu.make_async_copy(
                bufs.at[i % _NBUF], out_hbm.at[obase + i], osem
            ).start()
            return carry

        lax.fori_loop(0, _BPW, body, 0)

        for t in range(_BPW - _DIST, _BPW):
            pltpu.make_async_copy(
                bufs.at[t % _NBUF], out_hbm.at[obase + t], osem
            ).wait()

    return k(ptab, idx3d)


def kernel(actions, captions, action_emb, Wa, ba, caption_emb, Wc, bc):
    ptab = _project_tables(action_emb, Wa, ba, caption_emb, Wc, bc)
    idx = jnp.concatenate([actions, captions + ACT_VOCAB], axis=1)
    idx3d = idx.reshape(_NW, _BPW, TOK)
    enc = _gather_rows(ptab, idx3d)
    obs_mask = jnp.concatenate(
        [jnp.zeros((B, N_HIST), dtype=bool), jnp.ones((B, CAP_LEN), dtype=bool)],
        axis=1,
    )
    return enc, obs_mask


# speedup vs baseline: 7.4553x; 1.6933x over previous
import functools

import jax
import jax.numpy as jnp
from jax import lax
from jax.experimental import pallas as pl
from jax.experimental.pallas import tpu as pltpu
from jax.experimental.pallas import tpu_sc as plsc

B = 4096
N_HIST = 20
CAP_LEN = 50
TOK = N_HIST + CAP_LEN
ACT_VOCAB = 1000
CAP_VOCAB = 100000
DIM = 128

_CAP_BLK = 4000

_NC = 2
_NS = 16
_NW = _NC * _NS
_BPW = B // _NW
_NBUF = 6
_DIST = 3


def _proj_block(x_ref, w_ref, b_ref, o_ref):
    o_ref[...] = (
        jnp.dot(x_ref[...], w_ref[...], preferred_element_type=jnp.float32)
        + b_ref[...]
    )


def _project_actions(action_emb, Wa, ba):
    return pl.pallas_call(
        _proj_block,
        out_shape=jax.ShapeDtypeStruct((ACT_VOCAB, DIM), jnp.float32),
    )(action_emb, Wa, ba.reshape(1, DIM))


def _project_captions(caption_emb, Wc, bc):
    n_blocks = CAP_VOCAB // _CAP_BLK
    return pl.pallas_call(
        _proj_block,
        grid=(n_blocks,),
        in_specs=[
            pl.BlockSpec((_CAP_BLK, DIM), lambda i: (i, 0)),
            pl.BlockSpec((DIM, DIM), lambda i: (0, 0)),
            pl.BlockSpec((1, DIM), lambda i: (0, 0)),
        ],
        out_specs=pl.BlockSpec((_CAP_BLK, DIM), lambda i: (i, 0)),
        out_shape=jax.ShapeDtypeStruct((CAP_VOCAB, DIM), jnp.float32),
    )(caption_emb, Wc, bc.reshape(1, DIM))


def _ring_gather(tbl, idx_v, n, tbase, out_hbm, bufs, gsem, osem, cb):
    dist = min(_DIST, n)
    for t in range(dist):
        pltpu.make_async_copy(
            tbl.at[idx_v.at[t]], bufs.at[t % _NBUF], gsem
        ).start()

    def body(i, carry):
        @pl.when(i >= dist)
        def _():
            pltpu.make_async_copy(
                bufs.at[(i - dist) % _NBUF],
                out_hbm.at[tbase + i - dist, pl.ds(cb, _BPW)],
                osem,
            ).wait()

        @pl.when(i < n - dist)
        def _():
            pltpu.make_async_copy(
                tbl.at[idx_v.at[i + dist]],
                bufs.at[(i + dist) % _NBUF],
                gsem,
            ).start()

        pltpu.make_async_copy(
            tbl.at[idx_v.at[i]], bufs.at[i % _NBUF], gsem
        ).wait()
        pltpu.make_async_copy(
            bufs.at[i % _NBUF], out_hbm.at[tbase + i, pl.ds(cb, _BPW)], osem
        ).start()
        return carry

    lax.fori_loop(0, n, body, 0)

    for t in range(n - dist, n):
        pltpu.make_async_copy(
            bufs.at[t % _NBUF], out_hbm.at[tbase + t, pl.ds(cb, _BPW)], osem
        ).wait()


def _gather_rows(pa, pc, ia, ic):
    mesh = plsc.VectorSubcoreMesh(core_axis_name="c", subcore_axis_name="s")

    @functools.partial(
        pl.kernel,
        mesh=mesh,
        out_type=jax.ShapeDtypeStruct((TOK, B, DIM), jnp.float32),
        scratch_types=[
            pltpu.VMEM((N_HIST, _BPW), jnp.int32),
            pltpu.VMEM((CAP_LEN, _BPW), jnp.int32),
            pltpu.VMEM((_NBUF, _BPW, DIM), jnp.float32),
            pltpu.SemaphoreType.DMA,
            pltpu.SemaphoreType.DMA,
        ],
    )
    def k(pa_hbm, pc_hbm, ia_hbm, ic_hbm, out_hbm, ia_v, ic_v, bufs, gsem, osem):
        wid = lax.axis_index("s") * _NC + lax.axis_index("c")
        cb = wid * _BPW
        pltpu.sync_copy(ia_hbm.at[wid], ia_v)
        pltpu.sync_copy(ic_hbm.at[wid], ic_v)
        _ring_gather(pa_hbm, ia_v, N_HIST, 0, out_hbm, bufs, gsem, osem, cb)
        _ring_gather(pc_hbm, ic_v, CAP_LEN, N_HIST, out_hbm, bufs, gsem, osem, cb)

    return k(pa, pc, ia, ic)


def kernel(actions, captions, action_emb, Wa, ba, caption_emb, Wc, bc):
    pa = _project_actions(action_emb, Wa, ba)
    pc = _project_captions(caption_emb, Wc, bc)
    ia = actions.reshape(_NW, _BPW, N_HIST).transpose(0, 2, 1)
    ic = captions.reshape(_NW, _BPW, CAP_LEN).transpose(0, 2, 1)
    out_t = _gather_rows(pa, pc, ia, ic)
    enc = out_t.transpose(1, 0, 2)
    obs_mask = jnp.concatenate(
        [jnp.zeros((B, N_HIST), dtype=bool), jnp.ones((B, CAP_LEN), dtype=bool)],
        axis=1,
    )
    return enc, obs_mask
